# Initial kernel scaffold; baseline (speedup 1.0000x reference)
import jax
import jax.numpy as jnp
from jax.experimental import pallas as pl


def kernel(node_attr, edge_index, edge_attr, edge_sh, W1, b1, W2, b2):
    raise NotImplementedError("write your pallas kernel here")


# speedup vs baseline: 1.9655x; 1.9655x over previous
import functools

import jax
import jax.numpy as jnp
from jax import lax
from jax.experimental import pallas as pl
from jax.experimental.pallas import tpu as pltpu
import jax.experimental.pallas.tpu_sc as plsc

N_NODES = 10000
E = 160000
D_IN = 16
D_OUT = 32
D_EDGE = 16
HIDDEN = 16

NC = 2
NS = 16
NW = NC * NS
CHUNK = 128
CPW = 40
E_PAD = NW * CPW * CHUNK
WIDTH = 40
N_PAD = 10240
ROWS_PS = N_PAD // NS

def _sc_mesh():
  return plsc.VectorSubcoreMesh(
      core_axis_name="c", subcore_axis_name="s", num_cores=NC, num_subcores=NS)


def _gather_body(node_hbm, dst2_hbm, x1_hbm, idx_v, buf_v, sem):
  c = lax.axis_index("c")
  s = lax.axis_index("s")
  wid = s * NC + c
  pltpu.sync_copy(dst2_hbm.at[pl.ds(wid * CPW, CPW)], idx_v)

  def fire(j, _):
    pltpu.async_copy(node_hbm.at[idx_v.at[j]],
                     buf_v.at[pl.ds(j * CHUNK, CHUNK)], sem)
    return _

  lax.fori_loop(0, CPW, fire, 0, unroll=4)
  pltpu.make_async_copy(x1_hbm.at[pl.ds(0, CPW * CHUNK)], buf_v, sem).wait()
  pltpu.sync_copy(buf_v, x1_hbm.at[pl.ds(wid * CPW * CHUNK, CPW * CHUNK)])


def _gather(node_attr_pad, dst2):
  return pl.kernel(
      _gather_body,
      out_type=jax.ShapeDtypeStruct((E_PAD, D_IN), jnp.float32),
      mesh=_sc_mesh(),
      scratch_types=[
          pltpu.VMEM((CPW, CHUNK), jnp.int32),
          pltpu.VMEM((CPW * CHUNK, D_IN), jnp.float32),
          pltpu.SemaphoreType.DMA,
      ],
      compiler_params=pltpu.CompilerParams(use_tc_tiling_on_sc=False),
  )(node_attr_pad, dst2)


def _scatter_body(tpc_hbm, src2_hbm, zeros_hbm, out_hbm, idx_v, val_v, acc_sh):
  c = lax.axis_index("c")
  s = lax.axis_index("s")
  wid = s * NC + c
  pltpu.sync_copy(src2_hbm.at[pl.ds(wid * CPW, CPW)], idx_v)
  pltpu.sync_copy(zeros_hbm.at[pl.ds(s * ROWS_PS, ROWS_PS)],
                  acc_sh.at[pl.ds(s * ROWS_PS, ROWS_PS)])
  plsc.subcore_barrier()

  def step(j, _):
    row0 = (wid * CPW + j) * CHUNK
    pltpu.sync_copy(tpc_hbm.at[pl.ds(row0, CHUNK)], val_v)
    pltpu.sync_copy(val_v, acc_sh.at[idx_v.at[j]], add=True)
    return _

  lax.fori_loop(0, CPW, step, 0, unroll=1)
  plsc.subcore_barrier()
  pltpu.sync_copy(acc_sh.at[pl.ds(s * ROWS_PS, ROWS_PS)],
                  out_hbm.at[pl.ds(c * N_PAD + s * ROWS_PS, ROWS_PS)])


def _scatter(tpc, src2, zeros_np):
  return pl.kernel(
      _scatter_body,
      out_type=jax.ShapeDtypeStruct((NC * N_PAD, WIDTH), jnp.float32),
      mesh=_sc_mesh(),
      scratch_types=[
          pltpu.VMEM((CPW, CHUNK), jnp.int32),
          pltpu.VMEM((CHUNK, WIDTH), jnp.float32),
          pltpu.VMEM_SHARED((N_PAD, WIDTH), jnp.float32),
      ],
      compiler_params=pltpu.CompilerParams(use_tc_tiling_on_sc=False),
  )(tpc, src2, zeros_np)


def _mish(x):
  sp = jnp.maximum(x, 0.0) + jnp.log1p(jnp.exp(-jnp.abs(x)))
  return x * jnp.tanh(sp)


BE = 2048


def _dense_body(ea_ref, x1_ref, sm_ref, w1_ref, b1_ref, rr_ref, tt_ref,
                w2r_ref, b2r_ref, out_ref):
  ea = ea_ref[...]
  x1 = x1_ref[...]
  h = _mish(jnp.dot(ea, w1_ref[...], preferred_element_type=jnp.float32)
            + b1_ref[...])
  hrep = jnp.dot(h, rr_ref[...], preferred_element_type=jnp.float32)
  x1t = jnp.dot(x1, tt_ref[...], preferred_element_type=jnp.float32)
  tp0 = jnp.dot(hrep * x1t, w2r_ref[...], preferred_element_type=jnp.float32)
  tp0 = tp0 + jnp.dot(x1, b2r_ref[...], preferred_element_type=jnp.float32)
  sh = sm_ref[...][:, 0:1]
  mask = sm_ref[...][:, 1:2]
  tp = tp0 * (sh * 0.25)
  out_ref[...] = jnp.concatenate(
      [tp, mask, jnp.zeros((BE, WIDTH - D_OUT - 1), jnp.float32)], axis=1)


def _dense(ea, x1, sm, w1, b1, rr, tt, w2r, b2r):
  grid = (E_PAD // BE,)
  return pl.pallas_call(
      _dense_body,
      grid=grid,
      in_specs=[
          pl.BlockSpec((BE, D_EDGE), lambda i: (i, 0)),
          pl.BlockSpec((BE, D_IN), lambda i: (i, 0)),
          pl.BlockSpec((BE, 2), lambda i: (i, 0)),
          pl.BlockSpec((D_EDGE, HIDDEN), lambda i: (0, 0)),
          pl.BlockSpec((1, HIDDEN), lambda i: (0, 0)),
          pl.BlockSpec((HIDDEN, 256), lambda i: (0, 0)),
          pl.BlockSpec((D_IN, 256), lambda i: (0, 0)),
          pl.BlockSpec((256, D_OUT), lambda i: (0, 0)),
          pl.BlockSpec((D_IN, D_OUT), lambda i: (0, 0)),
      ],
      out_specs=pl.BlockSpec((BE, WIDTH), lambda i: (i, 0)),
      out_shape=jax.ShapeDtypeStruct((E_PAD, WIDTH), jnp.float32),
  )(ea, x1, sm, w1, b1, rr, tt, w2r, b2r)


BN = 512


def _combine_body(p0_ref, p1_ref, na_ref, out_ref):
  p0 = p0_ref[...]
  p1 = p1_ref[...]
  ssum = p0[:, :D_OUT] + p1[:, :D_OUT]
  cnt = p0[:, D_OUT:D_OUT + 1] + p1[:, D_OUT:D_OUT + 1]
  res = jnp.concatenate(
      [na_ref[...], jnp.zeros((BN, D_OUT - D_IN), jnp.float32)], axis=1)
  out_ref[...] = ssum / jnp.maximum(cnt, 1.0) + res


def _combine(p0, p1, na_pad):
  grid = (N_PAD // BN,)
  return pl.pallas_call(
      _combine_body,
      grid=grid,
      in_specs=[
          pl.BlockSpec((BN, WIDTH), lambda i: (i, 0)),
          pl.BlockSpec((BN, WIDTH), lambda i: (i, 0)),
          pl.BlockSpec((BN, D_IN), lambda i: (i, 0)),
      ],
      out_specs=pl.BlockSpec((BN, D_OUT), lambda i: (i, 0)),
      out_shape=jax.ShapeDtypeStruct((N_PAD, D_OUT), jnp.float32),
  )(p0, p1, na_pad)


@jax.jit
def kernel(node_attr, edge_index, edge_attr, edge_sh, W1, b1, W2, b2):
  edge_src = edge_index[0].astype(jnp.int32)
  edge_dst = edge_index[1].astype(jnp.int32)

  dst2 = jnp.zeros((E_PAD,), jnp.int32).at[:E].set(edge_dst)
  dst2 = dst2.reshape(E_PAD // CHUNK, CHUNK)
  src2 = jnp.zeros((E_PAD,), jnp.int32).at[:E].set(edge_src)
  src2 = src2.reshape(E_PAD // CHUNK, CHUNK)
  ea = jnp.zeros((E_PAD, D_EDGE), jnp.float32).at[:E].set(edge_attr)
  sm = jnp.zeros((E_PAD, 2), jnp.float32)
  sm = sm.at[:E, 0].set(edge_sh[:, 0])
  sm = sm.at[:E, 1].set(1.0)

  j = jnp.arange(256)
  rr = (j[None, :] // 16 == jnp.arange(HIDDEN)[:, None]).astype(jnp.float32)
  tt = (j[None, :] % 16 == jnp.arange(D_IN)[:, None]).astype(jnp.float32)
  w2r = W2.reshape(HIDDEN, D_IN, D_OUT).reshape(HIDDEN * D_IN, D_OUT)
  b2r = b2.reshape(D_IN, D_OUT)
  b1r = b1.reshape(1, HIDDEN)
  zeros_np = jnp.zeros((N_PAD, WIDTH), jnp.float32)
  na_pad = jnp.zeros((N_PAD, D_IN), jnp.float32).at[:N_NODES].set(node_attr)

  x1 = _gather(node_attr, dst2)
  tpc = _dense(ea, x1, sm, W1, b1r, rr, tt, w2r, b2r)
  partials = _scatter(tpc, src2, zeros_np)
  out = _combine(partials[:N_PAD], partials[N_PAD:], na_pad)
  return out[:N_NODES]
